# baseline (speedup 1.0000x reference)
import jax
import jax.numpy as jnp
from jax.experimental import pallas as pl
from jax.experimental.pallas import tpu as pltpu

ENCODING_DIM = 64


def _body(pos_s, num_s, table_hbm, out_v, buf_v, sem):
    s = pos_s[0] % num_s[0]
    g = s // 2
    r = s - g * 2
    cp = pltpu.make_async_copy(
        table_hbm.at[pl.ds(g * 2 * ENCODING_DIM, 2 * ENCODING_DIM)], buf_v, sem
    )
    cp.start()
    cp.wait()
    lo = buf_v[pl.ds(0, ENCODING_DIM)]
    hi = buf_v[pl.ds(ENCODING_DIM, ENCODING_DIM)]
    out_v[...] = jnp.where(r == 0, lo, hi)


def kernel(position, num_players, table):
    pos_arr = jnp.reshape(jnp.asarray(position, jnp.int32), (1,))
    num_arr = jnp.reshape(jnp.asarray(num_players, jnp.int32), (1,))
    flat = jnp.reshape(table, (-1,))
    out = pl.pallas_call(
        _body,
        in_specs=[
            pl.BlockSpec(memory_space=pltpu.SMEM),
            pl.BlockSpec(memory_space=pltpu.SMEM),
            pl.BlockSpec(memory_space=pl.ANY),
        ],
        out_specs=pl.BlockSpec(memory_space=pltpu.VMEM),
        out_shape=jax.ShapeDtypeStruct((ENCODING_DIM,), jnp.float32),
        scratch_shapes=[
            pltpu.VMEM((2 * ENCODING_DIM,), jnp.float32),
            pltpu.SemaphoreType.DMA,
        ],
    )(pos_arr, num_arr, flat)
    return out


# speedup vs baseline: 29.1690x; 29.1690x over previous
import jax
import jax.numpy as jnp
from jax import lax
from jax.experimental import pallas as pl
from jax.experimental.pallas import tpu as pltpu

ENCODING_DIM = 64
LANES = 128


def _body(pos_s, num_s, tableT_hbm, out_v, buf_v, sem):
    s = pos_s[0] % num_s[0]
    base = (s // LANES) * LANES
    r = s - base
    cp = pltpu.make_async_copy(tableT_hbm.at[:, pl.ds(base, LANES)], buf_v, sem)
    cp.start()
    cp.wait()
    lane = lax.broadcasted_iota(jnp.int32, (ENCODING_DIM, LANES), 1)
    sel = jnp.where(lane == r, buf_v[...], 0.0)
    out_v[...] = jnp.sum(sel, axis=1)


def kernel(position, num_players, table):
    pos_arr = jnp.reshape(jnp.asarray(position, jnp.int32), (1,))
    num_arr = jnp.reshape(jnp.asarray(num_players, jnp.int32), (1,))
    table_t = table.T
    out = pl.pallas_call(
        _body,
        in_specs=[
            pl.BlockSpec(memory_space=pltpu.SMEM),
            pl.BlockSpec(memory_space=pltpu.SMEM),
            pl.BlockSpec(memory_space=pl.ANY),
        ],
        out_specs=pl.BlockSpec(memory_space=pltpu.VMEM),
        out_shape=jax.ShapeDtypeStruct((ENCODING_DIM,), jnp.float32),
        scratch_shapes=[
            pltpu.VMEM((ENCODING_DIM, LANES), jnp.float32),
            pltpu.SemaphoreType.DMA,
        ],
    )(pos_arr, num_arr, table_t)
    return out
